# baseline (speedup 1.0000x reference)
import functools

import jax
import jax.numpy as jnp
from jax import lax
from jax.experimental import pallas as pl
from jax.experimental.pallas import tpu as pltpu
from jax.experimental.pallas import tpu_sc as plsc

NUM_CORES = 2
NUM_SUBCORES = 16
NUM_WORKERS = NUM_CORES * NUM_SUBCORES
LANES = 16
BATCH = 16384
DIM = 32
BPW = BATCH // NUM_WORKERS
CHUNK = 128
NCHUNK = BPW // CHUNK


def _gmf_body(uidx_hbm, iidx_hbm, utab_hbm, itab_hbm, wb_hbm, out_hbm,
              uidx_v, iidx_v, urows_v, irows_v, wb_v, out_v, sem_u, sem_i):
    c = lax.axis_index("c")
    s = lax.axis_index("s")
    wid = s * NUM_CORES + c
    base = pl.multiple_of(wid * BPW, BPW)

    pltpu.sync_copy(wb_hbm, wb_v)
    pltpu.sync_copy(uidx_hbm.at[wid], uidx_v)
    pltpu.sync_copy(iidx_hbm.at[wid], iidx_v)

    copies = []
    for j in range(NCHUNK):
        copies.append(pltpu.async_copy(
            utab_hbm.at[uidx_v.at[j]],
            urows_v.at[pl.ds(j * CHUNK, CHUNK)], sem_u))
        copies.append(pltpu.async_copy(
            itab_hbm.at[iidx_v.at[j]],
            irows_v.at[pl.ds(j * CHUNK, CHUNK)], sem_i))
    for cp in copies:
        cp.wait()

    lanes16 = lax.iota(jnp.int32, LANES)
    bias_v = wb_v[DIM, :]

    def group_body(g, carry):
        row0 = pl.multiple_of(g * LANES, LANES)
        rows = row0 + lanes16
        acc = bias_v
        for d in range(DIM):
            dv = jnp.full((LANES,), d, jnp.int32)
            uv = plsc.load_gather(urows_v, [rows, dv])
            iv = plsc.load_gather(irows_v, [rows, dv])
            wv = wb_v[d, :]
            acc = acc + uv * iv * wv
        out_v[pl.ds(row0, LANES)] = 1.0 / (1.0 + jnp.exp(-acc))
        return carry

    lax.fori_loop(0, BPW // LANES, group_body, 0)
    pltpu.sync_copy(out_v, out_hbm.at[pl.ds(base, BPW)])


@functools.partial(jax.jit, static_argnums=())
def _gmf_call(ui, ii, utab, itab, wb):
    mesh = plsc.VectorSubcoreMesh(core_axis_name="c", subcore_axis_name="s")
    f = functools.partial(
        pl.kernel,
        out_type=jax.ShapeDtypeStruct((BATCH,), jnp.float32),
        mesh=mesh,
        compiler_params=pltpu.CompilerParams(needs_layout_passes=False,
                                             use_tc_tiling_on_sc=False),
        scratch_types=[
            pltpu.VMEM((NCHUNK, CHUNK), jnp.int32),
            pltpu.VMEM((NCHUNK, CHUNK), jnp.int32),
            pltpu.VMEM((BPW, DIM), jnp.float32),
            pltpu.VMEM((BPW, DIM), jnp.float32),
            pltpu.VMEM((DIM + 1, LANES), jnp.float32),
            pltpu.VMEM((BPW,), jnp.float32),
            pltpu.SemaphoreType.DMA,
            pltpu.SemaphoreType.DMA,
        ],
    )(_gmf_body)
    return f(ui, ii, utab, itab, wb)


def kernel(user_indices, item_indices, user_table, item_table, affine_w, affine_b):
    ui = user_indices.astype(jnp.int32).reshape(NUM_WORKERS, NCHUNK, CHUNK)
    ii = item_indices.astype(jnp.int32).reshape(NUM_WORKERS, NCHUNK, CHUNK)
    wb = jnp.concatenate([
        jnp.broadcast_to(affine_w.reshape(DIM, 1), (DIM, LANES)),
        jnp.broadcast_to(affine_b.reshape(1, 1), (1, LANES)),
    ], axis=0).astype(jnp.float32)
    out = _gmf_call(ui, ii, user_table, item_table, wb)
    return out.reshape(BATCH, 1)
